# baseline (speedup 1.0000x reference)
import functools

import jax
import jax.numpy as jnp
from jax import lax
from jax.experimental import pallas as pl
from jax.experimental.pallas import tpu as pltpu
from jax.experimental.pallas import tpu_sc as plsc

_NUM_CORES = 2
_NUM_SUBCORES = 16
_NUM_WORKERS = _NUM_CORES * _NUM_SUBCORES
_LANES = 16
_NBUF = 2
_H0 = 128


def _emb_body(n, d, seq_per_w, n_pad, seq_base,
              xt_hbm, tbl_hbm, pos_hbm, out_hbm,
              xbuf_v, gidx_v, rows_v, obuf_v, pos_v,
              gsem0, gsem1, ssem0, ssem1):
  c = lax.axis_index("c")
  s = lax.axis_index("s")
  wid = s * _NUM_CORES + c
  base_seq = wid * seq_per_w
  gsems = (gsem0, gsem1)
  ssems = (ssem0, ssem1)
  n_outer = seq_per_w // _NBUF
  h1 = n - _H0
  nblk = (n + _LANES - 1) // _LANES
  iota = lax.iota(jnp.int32, _LANES)

  pltpu.sync_copy(pos_hbm, pos_v)
  pltpu.sync_copy(xt_hbm.at[:, pl.ds(seq_base + wid * seq_per_w, seq_per_w)],
                  xbuf_v.at[pl.ds(0, n)])

  def issue_gather(i_local, b):
    col = jnp.full((_LANES,), i_local, jnp.int32)
    for k in range(nblk):
      vals = plsc.load_gather(xbuf_v, [jnp.int32(_LANES * k) + iota, col])
      gidx_v.at[b][pl.ds(_LANES * k, _LANES)] = vals
    rows_b = rows_v.at[b]
    pltpu.async_copy(tbl_hbm.at[gidx_v.at[b, pl.ds(0, _H0)]],
                     rows_b.at[pl.ds(0, _H0)], gsems[b])
    pltpu.async_copy(tbl_hbm.at[gidx_v.at[b, pl.ds(_H0, h1)]],
                     rows_b.at[pl.ds(_H0, h1)], gsems[b])

  def drain_gather(b):
    pltpu.make_async_copy(tbl_hbm.at[pl.ds(0, n)], rows_v.at[b],
                          gsems[b]).wait()

  def out_dst(row0, nrows):
    return out_hbm.at[pl.ds(row0, nrows), pl.ds(0, d)]

  def drain_scatter(b):
    pltpu.make_async_copy(obuf_v.at[b], out_dst(0, n), ssems[b]).wait()

  for b in range(_NBUF):
    issue_gather(jnp.int32(b), b)

  @pl.loop(0, n_outer)
  def _outer(o):
    for b in range(_NBUF):
      i_local = o * _NBUF + b
      @pl.when(o >= 1)
      def _():
        drain_scatter(b)
      drain_gather(b)

      @plsc.parallel_loop(0, n, unroll=4)
      def _row(j):
        for k in range(d // _LANES):
          sl = pl.ds(k * _LANES, _LANES)
          obuf_v.at[b][j, sl] = rows_v.at[b][j, sl] + pos_v[j, sl]

      @pl.when(o < n_outer - 1)
      def _():
        issue_gather(i_local + _NBUF, b)
      pltpu.async_copy(obuf_v.at[b], out_dst((base_seq + i_local) * n, n),
                       ssems[b])

  for b in range(_NBUF):
    drain_scatter(b)


_TCB = 32768


def _tpose_body(d, tin_ref, tout_ref):
  tout_ref[:, :d] = jnp.transpose(tin_ref[...], (1, 0))


def _pad_transpose(tbl_t):
  d, v = tbl_t.shape
  grid = (v + _TCB - 1) // _TCB
  return pl.pallas_call(
      functools.partial(_tpose_body, d),
      grid=(grid,),
      in_specs=[pl.BlockSpec((d, _TCB), lambda i: (0, i))],
      out_specs=pl.BlockSpec((_TCB, 128), lambda i: (i, 0)),
      out_shape=jax.ShapeDtypeStruct((v, 128), jnp.float32),
  )(tbl_t)


def kernel(X, token_table, pos_table):
  b, n = X.shape
  v, d = token_table.shape
  assert b % (_NUM_WORKERS * _NBUF) == 0 and d % _LANES == 0
  seq_per_w = b // _NUM_WORKERS
  assert _H0 <= n < 2 * _H0
  n_pad = ((n + _LANES - 1) // _LANES) * _LANES

  xt = X.T.astype(jnp.int32)
  tbl128 = _pad_transpose(token_table.T)
  mesh = plsc.VectorSubcoreMesh(core_axis_name="c", subcore_axis_name="s")

  bh = b // 2
  seq_per_w_h = bh // _NUM_WORKERS

  def make_half(seq_base):
    return pl.kernel(
        functools.partial(_emb_body, n, d, seq_per_w_h, n_pad, seq_base),
        out_type=jax.ShapeDtypeStruct((bh * n, 128), jnp.float32),
        mesh=mesh,
        scratch_types=[
            pltpu.VMEM((n_pad, seq_per_w_h), jnp.int32),
            pltpu.VMEM((_NBUF, n_pad), jnp.int32),
            pltpu.VMEM((_NBUF, n, 128), jnp.float32),
            pltpu.VMEM((_NBUF, n, d), jnp.float32),
            pltpu.VMEM((n, d), jnp.float32),
            pltpu.SemaphoreType.DMA,
            pltpu.SemaphoreType.DMA,
            pltpu.SemaphoreType.DMA,
            pltpu.SemaphoreType.DMA,
        ],
        compiler_params=pltpu.CompilerParams(use_tc_tiling_on_sc=False,
                                             needs_layout_passes=False),
    )

  o1 = make_half(0)(xt, tbl128, pos_table)
  o2 = make_half(bh)(xt, tbl128, pos_table)
  return jnp.concatenate(
      [o1[:, :d].reshape(bh, n, d), o2[:, :d].reshape(bh, n, d)], axis=0)


# speedup vs baseline: 1.1377x; 1.1377x over previous
import functools

import jax
import jax.numpy as jnp
from jax import lax
from jax.experimental import pallas as pl
from jax.experimental.pallas import tpu as pltpu
from jax.experimental.pallas import tpu_sc as plsc

_NUM_CORES = 2
_NUM_SUBCORES = 16
_NUM_WORKERS = _NUM_CORES * _NUM_SUBCORES
_LANES = 16
_NBUF = 3
_H0 = 128


def _emb_body(n, d, seq_per_w, n_pad,
              xt_hbm, tbl_hbm, pos_hbm, out_hbm,
              xbuf_v, gidx_v, rows_v, pos_v,
              gsem0, gsem1, gsem2, ssem0, ssem1, ssem2):
  c = lax.axis_index("c")
  s = lax.axis_index("s")
  wid = s * _NUM_CORES + c
  base_seq = wid * seq_per_w
  gsems = (gsem0, gsem1, gsem2)
  ssems = (ssem0, ssem1, ssem2)
  h1 = n - _H0
  nblk = (n + _LANES - 1) // _LANES
  iota = lax.iota(jnp.int32, _LANES)
  main = (seq_per_w - 2) // _NBUF * _NBUF

  pltpu.sync_copy(pos_hbm, pos_v)
  pltpu.sync_copy(xt_hbm.at[:, pl.ds(wid * seq_per_w, seq_per_w)],
                  xbuf_v.at[pl.ds(0, n)])

  def issue_gather(i_local, b):
    col = jnp.full((_LANES,), i_local, jnp.int32)
    for k in range(nblk):
      vals = plsc.load_gather(xbuf_v, [jnp.int32(_LANES * k) + iota, col])
      gidx_v.at[b][pl.ds(_LANES * k, _LANES)] = vals
    rows_b = rows_v.at[b]
    pltpu.async_copy(tbl_hbm.at[gidx_v.at[b, pl.ds(0, _H0)]],
                     rows_b.at[pl.ds(0, _H0)], gsems[b])
    pltpu.async_copy(tbl_hbm.at[gidx_v.at[b, pl.ds(_H0, h1)]],
                     rows_b.at[pl.ds(_H0, h1)], gsems[b])

  def drain_gather(b):
    pltpu.make_async_copy(tbl_hbm.at[pl.ds(0, n)], rows_v.at[b],
                          gsems[b]).wait()

  def out_dst(row0, nrows):
    return out_hbm.at[pl.ds(row0, nrows), pl.ds(0, d)]

  def out_src(b):
    return rows_v.at[b].at[:, pl.ds(0, d)]

  def drain_scatter(b):
    pltpu.make_async_copy(out_src(b), out_dst(0, n), ssems[b]).wait()

  def step(i_local, b, drain_prev, prefetch):
    drain_gather(b)

    @plsc.parallel_loop(0, n, unroll=4)
    def _row(j):
      for k in range(d // _LANES):
        sl = pl.ds(k * _LANES, _LANES)
        plsc.addupdate(rows_v.at[b].at[j, sl], pos_v[j, sl])

    pltpu.async_copy(out_src(b), out_dst((base_seq + i_local) * n, n),
                     ssems[b])
    bp = (b + 2) % _NBUF
    if drain_prev:
      @pl.when(i_local >= 1)
      def _():
        drain_scatter(bp)
    if prefetch:
      issue_gather(i_local + 2, bp)

  for b in range(2):
    issue_gather(jnp.int32(b), b)

  @pl.loop(0, main, step=_NBUF)
  def _outer(o):
    for bb in range(_NBUF):
      step(o + bb, bb, drain_prev=True, prefetch=True)

  for i in range(main, seq_per_w):
    step(jnp.int32(i), i % _NBUF, drain_prev=True, prefetch=False)
  drain_scatter((seq_per_w - 1) % _NBUF)


_TCB = 32768


def _tpose_body(d, tin_ref, tout_ref):
  tout_ref[:, :d] = jnp.transpose(tin_ref[...], (1, 0))


def _pad_transpose(tbl_t):
  d, v = tbl_t.shape
  grid = (v + _TCB - 1) // _TCB
  return pl.pallas_call(
      functools.partial(_tpose_body, d),
      grid=(grid,),
      in_specs=[pl.BlockSpec((d, _TCB), lambda i: (0, i))],
      out_specs=pl.BlockSpec((_TCB, 128), lambda i: (i, 0)),
      out_shape=jax.ShapeDtypeStruct((v, 128), jnp.float32),
  )(tbl_t)


def kernel(X, token_table, pos_table):
  b, n = X.shape
  v, d = token_table.shape
  assert b % _NUM_WORKERS == 0 and d % _LANES == 0
  seq_per_w = b // _NUM_WORKERS
  assert _H0 <= n < 2 * _H0 and seq_per_w > _NBUF + 2
  n_pad = ((n + _LANES - 1) // _LANES) * _LANES

  xt = X.T.astype(jnp.int32)
  tbl128 = _pad_transpose(token_table.T)
  mesh = plsc.VectorSubcoreMesh(core_axis_name="c", subcore_axis_name="s")

  emb = pl.kernel(
      functools.partial(_emb_body, n, d, seq_per_w, n_pad),
      out_type=jax.ShapeDtypeStruct((b * n, 128), jnp.float32),
      mesh=mesh,
      scratch_types=[
          pltpu.VMEM((n_pad, seq_per_w), jnp.int32),
          pltpu.VMEM((_NBUF, n_pad), jnp.int32),
          pltpu.VMEM((_NBUF, n, 128), jnp.float32),
          pltpu.VMEM((n, d), jnp.float32),
          pltpu.SemaphoreType.DMA,
          pltpu.SemaphoreType.DMA,
          pltpu.SemaphoreType.DMA,
          pltpu.SemaphoreType.DMA,
          pltpu.SemaphoreType.DMA,
          pltpu.SemaphoreType.DMA,
      ],
      compiler_params=pltpu.CompilerParams(use_tc_tiling_on_sc=False,
                                           needs_layout_passes=False),
  )
  out = emb(xt, tbl128, pos_table)
  return out[:, :d].reshape(b, n, d)


# speedup vs baseline: 1.2165x; 1.0693x over previous
import functools

import jax
import jax.numpy as jnp
from jax import lax
from jax.experimental import pallas as pl
from jax.experimental.pallas import tpu as pltpu
from jax.experimental.pallas import tpu_sc as plsc

_NUM_CORES = 2
_NUM_SUBCORES = 16
_NUM_WORKERS = _NUM_CORES * _NUM_SUBCORES
_LANES = 16
_NBUF = 2
_H0 = 128


def _emb_body(n, d, seq_per_w, n_pad,
              xt_hbm, tbl_hbm, pos_hbm, out_hbm,
              xbuf_v, gidx_v, rows_v, obuf_v, pos_v,
              gsem0, gsem1, ssem0, ssem1):
  c = lax.axis_index("c")
  s = lax.axis_index("s")
  wid = s * _NUM_CORES + c
  base_seq = wid * seq_per_w
  gsems = (gsem0, gsem1)
  ssems = (ssem0, ssem1)
  n_outer = seq_per_w // _NBUF
  h1 = n - _H0
  nblk = (n + _LANES - 1) // _LANES
  iota = lax.iota(jnp.int32, _LANES)

  pltpu.sync_copy(pos_hbm, pos_v)
  pltpu.sync_copy(xt_hbm.at[:, pl.ds(wid * seq_per_w, seq_per_w)],
                  xbuf_v.at[pl.ds(0, n)])

  def issue_gather(i_local, b):
    col = jnp.full((_LANES,), i_local, jnp.int32)
    for k in range(nblk):
      vals = plsc.load_gather(xbuf_v, [jnp.int32(_LANES * k) + iota, col])
      gidx_v.at[b][pl.ds(_LANES * k, _LANES)] = vals
    rows_b = rows_v.at[b]
    pltpu.async_copy(tbl_hbm.at[gidx_v.at[b, pl.ds(0, _H0)]],
                     rows_b.at[pl.ds(0, _H0)], gsems[b])
    pltpu.async_copy(tbl_hbm.at[gidx_v.at[b, pl.ds(_H0, h1)]],
                     rows_b.at[pl.ds(_H0, h1)], gsems[b])

  def drain_gather(b):
    pltpu.make_async_copy(tbl_hbm.at[pl.ds(0, n)], rows_v.at[b],
                          gsems[b]).wait()

  def out_dst(row0, nrows):
    return out_hbm.at[pl.ds(row0, nrows), pl.ds(0, d)]

  def drain_scatter(b):
    pltpu.make_async_copy(obuf_v.at[b], out_dst(0, n), ssems[b]).wait()

  for b in range(_NBUF):
    issue_gather(jnp.int32(b), b)

  @pl.loop(0, n_outer)
  def _outer(o):
    for b in range(_NBUF):
      i_local = o * _NBUF + b
      @pl.when(o >= 1)
      def _():
        drain_scatter(b)
      drain_gather(b)

      @plsc.parallel_loop(0, n, unroll=4)
      def _row(j):
        for k in range(d // _LANES):
          sl = pl.ds(k * _LANES, _LANES)
          obuf_v.at[b][j, sl] = rows_v.at[b][j, sl] + pos_v[j, sl]

      @pl.when(o < n_outer - 1)
      def _():
        issue_gather(i_local + _NBUF, b)
      pltpu.async_copy(obuf_v.at[b], out_dst((base_seq + i_local) * n, n),
                       ssems[b])

  for b in range(_NBUF):
    drain_scatter(b)


_TCB = 32768


def _tpose_body(d, tin_ref, tout_ref):
  tout_ref[:, :d] = jnp.transpose(tin_ref[...], (1, 0))


def _pad_transpose(tbl_t):
  d, v = tbl_t.shape
  grid = (v + _TCB - 1) // _TCB
  return pl.pallas_call(
      functools.partial(_tpose_body, d),
      grid=(grid,),
      in_specs=[pl.BlockSpec((d, _TCB), lambda i: (0, i))],
      out_specs=pl.BlockSpec((_TCB, 128), lambda i: (i, 0)),
      out_shape=jax.ShapeDtypeStruct((v, 128), jnp.float32),
  )(tbl_t)


def kernel(X, token_table, pos_table):
  b, n = X.shape
  v, d = token_table.shape
  assert b % (_NUM_WORKERS * _NBUF) == 0 and d % _LANES == 0
  seq_per_w = b // _NUM_WORKERS
  assert _H0 <= n < 2 * _H0
  n_pad = ((n + _LANES - 1) // _LANES) * _LANES

  xt = X.T.astype(jnp.int32)
  tbl128 = _pad_transpose(token_table.T)
  mesh = plsc.VectorSubcoreMesh(core_axis_name="c", subcore_axis_name="s")

  emb = pl.kernel(
      functools.partial(_emb_body, n, d, seq_per_w, n_pad),
      out_type=jax.ShapeDtypeStruct((b * n, 128), jnp.float32),
      mesh=mesh,
      scratch_types=[
          pltpu.VMEM((n_pad, seq_per_w), jnp.int32),
          pltpu.VMEM((_NBUF, n_pad), jnp.int32),
          pltpu.VMEM((_NBUF, n, 128), jnp.float32),
          pltpu.VMEM((_NBUF, n, d), jnp.float32),
          pltpu.VMEM((n, d), jnp.float32),
          pltpu.SemaphoreType.DMA,
          pltpu.SemaphoreType.DMA,
          pltpu.SemaphoreType.DMA,
          pltpu.SemaphoreType.DMA,
      ],
      compiler_params=pltpu.CompilerParams(use_tc_tiling_on_sc=False,
                                           needs_layout_passes=False),
  )
  out = emb(xt, tbl128, pos_table)
  return out[:, :d].reshape(b, n, d)
